# baseline (speedup 1.0000x reference)
import jax
import jax.numpy as jnp
from jax.experimental import pallas as pl
from jax.experimental.pallas import tpu as pltpu

N = 4096
N1 = 2048
N2 = 512
K = 128
R1, R2 = 0.2, 0.5
NQ = 4096


def _mm(a, b):
    return jax.lax.dot_general(a, b, (((1,), (0,)), ((), ())))


def _fps_body(pos_ref, idx_ref):
    n = pos_ref.shape[1]
    n_samples = idx_ref.shape[0]
    px = pos_ref[0:1, :]
    py = pos_ref[1:2, :]
    pz = pos_ref[2:3, :]
    lane = jax.lax.broadcasted_iota(jnp.int32, (1, n), 1)

    idx_ref[0] = 0
    m0 = (lane == 0).astype(jnp.float32)
    lx0 = jnp.sum(px * m0)
    ly0 = jnp.sum(py * m0)
    lz0 = jnp.sum(pz * m0)
    dists0 = jnp.full((1, n), jnp.inf, dtype=jnp.float32)

    def body(i, state):
        lx, ly, lz, dists = state
        d = (px - lx) ** 2 + (py - ly) ** 2 + (pz - lz) ** 2
        dists = jnp.minimum(dists, d)
        nxt = jnp.argmax(dists).astype(jnp.int32)
        idx_ref[i] = nxt
        m = (lane == nxt).astype(jnp.float32)
        return (jnp.sum(px * m), jnp.sum(py * m), jnp.sum(pz * m), dists)

    jax.lax.fori_loop(1, n_samples, body, (lx0, ly0, lz0, dists0))


def _fps(pos, n_samples):
    n = pos.shape[0]
    pos_t = jnp.zeros((4, n), jnp.float32).at[:3, :].set(pos.T)
    return pl.pallas_call(
        _fps_body,
        in_specs=[pl.BlockSpec(memory_space=pltpu.VMEM)],
        out_specs=pl.BlockSpec(memory_space=pltpu.SMEM),
        out_shape=jax.ShapeDtypeStruct((n_samples,), jnp.int32),
    )(pos_t)


def _radius_nn(pos_src, pos_q, r, k):
    d2 = jnp.sum((pos_q[:, None, :] - pos_src[None, :, :]) ** 2, axis=-1)
    neg = jnp.where(d2 <= r * r, -d2, -jnp.inf)
    vals, nbr = jax.lax.top_k(neg, k)
    valid = vals > -jnp.inf
    return jnp.where(valid, nbr, 0).astype(jnp.int32), valid


def _sa_kernel(fa_ref, fb_ref, m_ref,
               w1a_ref, w1b_ref, b1_ref, g1_ref, e1_ref,
               w2_ref, b2_ref, g2_ref, e2_ref,
               w3_ref, b3_ref, o_ref,
               s1, q1, s2, q2, cnt):
    p = pl.program_id(0)
    b = pl.program_id(1)
    mask = m_ref[...]

    def lin1():
        return _mm(fa_ref[...], w1a_ref[...]) + _mm(fb_ref[...], w1b_ref[...]) + b1_ref[...]

    @pl.when(jnp.logical_and(p == 0, b == 0))
    def _():
        s1[...] = jnp.zeros_like(s1)
        cnt[...] = jnp.zeros_like(cnt)

    @pl.when(p == 0)
    def _():
        h1 = lin1()
        s1[...] += jnp.sum(h1 * mask, axis=0, keepdims=True)
        cnt[...] += jnp.sum(mask)

    @pl.when(jnp.logical_and(p == 1, b == 0))
    def _():
        q1[...] = jnp.zeros_like(q1)

    @pl.when(p == 1)
    def _():
        h1 = lin1()
        c = h1 - s1[...] / cnt[0, 0]
        q1[...] += jnp.sum(c * c * mask, axis=0, keepdims=True)

    def affine1(h1):
        n = cnt[0, 0]
        mean = s1[...] / n
        sc = g1_ref[...] / jnp.sqrt(q1[...] / n + 1e-5)
        return jax.nn.relu((h1 - mean) * sc + e1_ref[...])

    def lin2():
        return _mm(affine1(lin1()), w2_ref[...]) + b2_ref[...]

    @pl.when(jnp.logical_and(p == 2, b == 0))
    def _():
        s2[...] = jnp.zeros_like(s2)

    @pl.when(p == 2)
    def _():
        s2[...] += jnp.sum(lin2() * mask, axis=0, keepdims=True)

    @pl.when(jnp.logical_and(p == 3, b == 0))
    def _():
        q2[...] = jnp.zeros_like(q2)

    @pl.when(p == 3)
    def _():
        c = lin2() - s2[...] / cnt[0, 0]
        q2[...] += jnp.sum(c * c * mask, axis=0, keepdims=True)

    @pl.when(p == 4)
    def _():
        h2 = lin2()
        n = cnt[0, 0]
        mean = s2[...] / n
        sc = g2_ref[...] / jnp.sqrt(q2[...] / n + 1e-5)
        a2 = jax.nn.relu((h2 - mean) * sc + e2_ref[...])
        h3 = _mm(a2, w3_ref[...]) + b3_ref[...]
        h3 = jnp.where(mask > 0, h3, -1e10)
        nc = o_ref.shape[0]
        fo = o_ref.shape[1]
        o_ref[...] = jnp.max(h3.reshape(nc, K, fo), axis=1)


def _sa_stage(feat_a, feat_b, valid, layers, centers_per_block):
    (W1, b1, g1, e1), (W2, b2, g2, e2), (W3, b3, _, _) = layers
    ncent = valid.shape[0]
    f_in_a = feat_a.shape[-1]
    f_in_b = feat_b.shape[-1]
    f1 = W1.shape[1]
    f2 = W2.shape[1]
    f3 = W3.shape[1]
    E = ncent * K
    eb = centers_per_block * K
    nb = ncent // centers_per_block
    fa = feat_a.reshape(E, f_in_a)
    fb = feat_b.reshape(E, f_in_b)
    m = valid.reshape(E, 1).astype(jnp.float32)
    zero = lambda p, b: (0, 0)
    return pl.pallas_call(
        _sa_kernel,
        grid=(5, nb),
        in_specs=[
            pl.BlockSpec((eb, f_in_a), lambda p, b: (b, 0)),
            pl.BlockSpec((eb, f_in_b), lambda p, b: (b, 0)),
            pl.BlockSpec((eb, 1), lambda p, b: (b, 0)),
            pl.BlockSpec((f_in_a, f1), zero),
            pl.BlockSpec((f_in_b, f1), zero),
            pl.BlockSpec((1, f1), zero),
            pl.BlockSpec((1, f1), zero),
            pl.BlockSpec((1, f1), zero),
            pl.BlockSpec((f1, f2), zero),
            pl.BlockSpec((1, f2), zero),
            pl.BlockSpec((1, f2), zero),
            pl.BlockSpec((1, f2), zero),
            pl.BlockSpec((f2, f3), zero),
            pl.BlockSpec((1, f3), zero),
        ],
        out_specs=pl.BlockSpec((centers_per_block, f3), lambda p, b: (b, 0)),
        out_shape=jax.ShapeDtypeStruct((ncent, f3), jnp.float32),
        scratch_shapes=[
            pltpu.VMEM((1, f1), jnp.float32),
            pltpu.VMEM((1, f1), jnp.float32),
            pltpu.VMEM((1, f2), jnp.float32),
            pltpu.VMEM((1, f2), jnp.float32),
            pltpu.VMEM((1, 1), jnp.float32),
        ],
    )(fa, fb, m,
      W1[:f_in_a], W1[f_in_a:], b1[None], g1[None], e1[None],
      W2, b2[None], g2[None], e2[None],
      W3, b3[None])


def _sa3_kernel(x2_ref, p2_ref,
                w1a_ref, w1b_ref, b1_ref, g1_ref, e1_ref,
                w2_ref, b2_ref, g2_ref, e2_ref,
                w3_ref, b3_ref,
                we_ref, be_ref, wm_ref, bm_ref, wl_ref, bl_ref,
                mu_ref, lv_ref):
    h = _mm(x2_ref[...], w1a_ref[...]) + _mm(p2_ref[...], w1b_ref[...]) + b1_ref[...]
    mean = jnp.mean(h, axis=0, keepdims=True)
    var = jnp.mean((h - mean) ** 2, axis=0, keepdims=True)
    h = jax.nn.relu(g1_ref[...] * (h - mean) / jnp.sqrt(var + 1e-5) + e1_ref[...])
    h = _mm(h, w2_ref[...]) + b2_ref[...]
    mean = jnp.mean(h, axis=0, keepdims=True)
    var = jnp.mean((h - mean) ** 2, axis=0, keepdims=True)
    h = jax.nn.relu(g2_ref[...] * (h - mean) / jnp.sqrt(var + 1e-5) + e2_ref[...])
    h = _mm(h, w3_ref[...]) + b3_ref[...]
    xg = jnp.max(h, axis=0, keepdims=True)
    enc = _mm(xg, we_ref[...]) + be_ref[...]
    mu = _mm(enc, wm_ref[...]) + bm_ref[...]
    lv = _mm(enc, wl_ref[...]) + bl_ref[...]
    mu_ref[...] = mu
    lv_ref[...] = lv


def _sa3_vae(x2, pos2, sa3, enc_l, mu_l, lv_l, eps):
    (W1, b1, g1, e1), (W2, b2, g2, e2), (W3, b3, _, _) = sa3
    (We, be, _, _) = enc_l[0]
    (Wm, bm, _, _) = mu_l[0]
    (Wl, bl, _, _) = lv_l[0]
    f2 = x2.shape[1]
    vspec = pl.BlockSpec(memory_space=pltpu.VMEM)
    out = pl.pallas_call(
        _sa3_kernel,
        in_specs=[vspec] * 19,
        out_specs=[vspec] * 2,
        out_shape=[jax.ShapeDtypeStruct((1, 512), jnp.float32)] * 2,
    )(x2, pos2,
      W1[:f2], W1[f2:], b1[None], g1[None], e1[None],
      W2, b2[None], g2[None], e2[None],
      W3, b3[None],
      We, be[None], Wm, bm[None], Wl, bl[None])
    mu, lv = out
    z = mu + eps * jnp.exp(0.5 * lv)
    return mu, lv, z


def _sdf_block_kernel(z_ref, q_ref, w1a_ref, w1b_ref, b1_ref, w2_ref, b2_ref,
                      w3a_ref, w3b_ref, b3_ref, w4_ref, b4_ref, o_ref):
    q = q_ref[...]
    z = z_ref[...]
    h = _mm(z, w1a_ref[...]) + _mm(q, w1b_ref[...]) + b1_ref[...]
    h = jax.nn.relu(h)
    h = _mm(h, w2_ref[...]) + b2_ref[...]
    h2 = _mm(h, w3a_ref[...]) + _mm(q, w3b_ref[...]) + b3_ref[...]
    h2 = jax.nn.relu(h2)
    o_ref[...] = _mm(h2, w4_ref[...]) + b4_ref[...]


def _sdf_decode(z, query_pos, sdf1, sdf2):
    (W1, b1, _, _), (W2, b2, _, _) = sdf1
    (W3, b3, _, _), (W4, b4, _, _) = sdf2
    B = 512
    grid = NQ // B
    out = pl.pallas_call(
        _sdf_block_kernel,
        grid=(grid,),
        in_specs=[
            pl.BlockSpec((1, 512), lambda i: (0, 0)),
            pl.BlockSpec((B, 3), lambda i: (i, 0)),
            pl.BlockSpec((512, 256), lambda i: (0, 0)),
            pl.BlockSpec((3, 256), lambda i: (0, 0)),
            pl.BlockSpec((1, 256), lambda i: (0, 0)),
            pl.BlockSpec((256, 128), lambda i: (0, 0)),
            pl.BlockSpec((1, 128), lambda i: (0, 0)),
            pl.BlockSpec((128, 64), lambda i: (0, 0)),
            pl.BlockSpec((3, 64), lambda i: (0, 0)),
            pl.BlockSpec((1, 64), lambda i: (0, 0)),
            pl.BlockSpec((64, 1), lambda i: (0, 0)),
            pl.BlockSpec((1, 1), lambda i: (0, 0)),
        ],
        out_specs=pl.BlockSpec((B, 1), lambda i: (i, 0)),
        out_shape=jax.ShapeDtypeStruct((NQ, 1), jnp.float32),
    )(z, query_pos, W1[:512], W1[512:], b1[None], W2, b2[None],
      W3[:128], W3[128:], b3[None], W4, b4[None])
    return jnp.tanh(out)


_XLA_SA = True


def _bn_x(h, g, be, valid=None):
    if valid is None:
        axes = tuple(range(h.ndim - 1))
        mean = jnp.mean(h, axis=axes)
        var = jnp.var(h, axis=axes)
    else:
        w = valid[..., None].astype(h.dtype)
        n = jnp.sum(w)
        mean = jnp.sum(h * w, axis=(0, 1)) / n
        var = jnp.sum(((h - mean) ** 2) * w, axis=(0, 1)) / n
    return g * (h - mean) / jnp.sqrt(var + 1e-5) + be


def _apply_mlp_x(h, layers, valid=None):
    n = len(layers)
    for i, (W, b, g, be) in enumerate(layers):
        h = h @ W + b
        if i < n - 1:
            if g is not None:
                h = _bn_x(h, g, be, valid)
            h = jax.nn.relu(h)
    return h


def kernel(x, pos, batch, query_pos, params, eps):
    idx1 = _fps(pos, N1)
    pos1 = pos[idx1]
    nbr1, val1 = _radius_nn(pos, pos1, R1, K)
    idx2 = _fps(pos1, N2)
    pos2 = pos1[idx2]
    nbr2, val2 = _radius_nn(pos1, pos2, R2, K)

    fa1 = x[nbr1]
    fb1 = pos[nbr1] - pos1[:, None, :]
    if _XLA_SA:
        h = _apply_mlp_x(jnp.concatenate([fa1, fb1], axis=-1), params['sa1'], val1)
        x1 = jnp.max(jnp.where(val1[..., None], h, -1e10), axis=1)
    else:
        x1 = _sa_stage(fa1, fb1, val1, params['sa1'], centers_per_block=16)

    fa2 = x1[nbr2]
    fb2 = pos1[nbr2] - pos2[:, None, :]
    if _XLA_SA:
        h2 = _apply_mlp_x(jnp.concatenate([fa2, fb2], axis=-1), params['sa2'], val2)
        x2 = jnp.max(jnp.where(val2[..., None], h2, -1e10), axis=1)
    else:
        x2 = _sa_stage(fa2, fb2, val2, params['sa2'], centers_per_block=16)

    if _XLA_SA:
        h3 = _apply_mlp_x(jnp.concatenate([x2, pos2], axis=1), params['sa3'])
        xg = jnp.max(h3, axis=0, keepdims=True)
        enc = _apply_mlp_x(xg, params['enc'])
        mu = _apply_mlp_x(enc, params['mu'])
        logvar = _apply_mlp_x(enc, params['lv'])
        z = mu + eps * jnp.exp(0.5 * logvar)
    else:
        mu, logvar, z = _sa3_vae(x2, pos2, params['sa3'], params['enc'],
                                 params['mu'], params['lv'], eps)
    out = _sdf_decode(z, query_pos, params['sdf1'], params['sdf2'])
    return out, mu, logvar
